# baseline (speedup 1.0000x reference)
import functools

import jax
import jax.numpy as jnp
from jax import lax
from jax.experimental import pallas as pl
from jax.experimental.pallas import tpu as pltpu
from jax.experimental.pallas import tpu_sc as plsc

NUM_EXPERTS = 64
TOP_K = 2
T = 32768
D_MODEL = 768
NSLOT = T * TOP_K
NC = 2
NS = 16
NW = NC * NS
BLK = NSLOT // NW
VPB = BLK // 16
ROWS_PER_CHUNK = 64
CHUNKS = BLK // ROWS_PER_CHUNK


def _make_mesh():
    return plsc.VectorSubcoreMesh(core_axis_name="c", subcore_axis_name="s")


def _sort_kernel(experts_hbm, gates_hbm, tok_out, gates_out, tpe_out, hists_hbm,
                 ea, eb, hist2, hists, run, dest_buf, tokv, gatesv,
                 scat_sems):
    c = lax.axis_index("c")
    s = lax.axis_index("s")
    q = c * NS + s

    ones = jnp.ones((16,), jnp.int32)
    zv = jnp.zeros((16,), jnp.int32)

    pltpu.sync_copy(experts_hbm.at[pl.ds(s * 2 * BLK, 2 * BLK)], ea)
    for bl in range(2):
        for k in range(4):
            hist2[bl, pl.ds(k * 16, 16)] = zv

        def hist_body(i, carry, bl=bl):
            v = ea[pl.ds(bl * BLK + i * 16, 16)]
            plsc.addupdate_scatter(hist2.at[bl], [v], ones)
            return carry

        lax.fori_loop(0, VPB, hist_body, 0)
    pltpu.sync_copy(hist2, hists_hbm.at[pl.ds(c * NW + 2 * s, 2)])
    plsc.subcore_barrier()
    pltpu.sync_copy(hists_hbm.at[pl.ds(c * NW, NW)], hists)

    carry = jnp.zeros((), jnp.int32)
    for k in range(4):
        tot_k = zv
        pre_k = zv
        for b in range(32):
            row = hists[b, pl.ds(k * 16, 16)]
            tot_k = tot_k + row
            pre_k = pre_k + jnp.where(q > b, row, zv)
        incl = plsc.cumsum(tot_k)
        base_k = incl - tot_k + carry + pre_k
        run[pl.ds(k * 16, 16)] = base_k
        carry = carry + jnp.sum(tot_k)
        hist2[0, pl.ds(k * 16, 16)] = tot_k

    @pl.when(jnp.logical_and(c == 0, s == 0))
    def _():
        pltpu.sync_copy(hist2.at[0], tpe_out)

    pltpu.sync_copy(experts_hbm.at[pl.ds(q * BLK, BLK)], eb)
    pltpu.sync_copy(gates_hbm.at[pl.ds(q * BLK, BLK)], gatesv)
    iota16 = lax.iota(jnp.int32, 16)
    for j2 in range(16):
        for jj in range(8):
            j = j2 * 8 + jj
            v = eb[pl.ds(j * 16, 16)]
            b = plsc.load_gather(run, [v])
            r, _ = plsc.scan_count(v)
            dest = b + r - 1
            plsc.addupdate_scatter(run, [v], ones)
            dest_buf[j2, pl.ds(jj * 16, 16)] = dest
            slot0 = q * BLK + j * 16
            tokv[pl.ds(j * 16, 16)] = lax.shift_right_logical(slot0 + iota16, 1)

    descs = []
    for j2 in range(16):
        idx = dest_buf.at[j2]
        descs.append(pltpu.async_copy(
            tokv.at[pl.ds(j2 * 128, 128)], tok_out.at[idx], scat_sems[0]))
        descs.append(pltpu.async_copy(
            gatesv.at[pl.ds(j2 * 128, 128)], gates_out.at[idx], scat_sems[1]))
    for d in descs:
        d.wait()


def _gather_kernel(hidden_hbm, tokidx_hbm, out_hbm, idxv, bufs, gsems, osems):
    c = lax.axis_index("c")
    s = lax.axis_index("s")
    q = c * NS + s
    base_row = q * BLK

    pltpu.sync_copy(tokidx_hbm.at[pl.ds(base_row, BLK)], idxv)

    def start_gather(g):
        b = g % 2
        idx = idxv.at[pl.ds(g * ROWS_PER_CHUNK, ROWS_PER_CHUNK)]
        return pltpu.async_copy(hidden_hbm.at[idx], bufs.at[b], gsems[b])

    g_descs = {}
    o_descs = {}
    g_descs[0] = start_gather(0)
    for g in range(CHUNKS):
        b = g % 2
        if g + 1 < CHUNKS:
            if g - 1 >= 0:
                o_descs[g - 1].wait()
            g_descs[g + 1] = start_gather(g + 1)
        g_descs[g].wait()
        o_descs[g] = pltpu.async_copy(
            bufs.at[b],
            out_hbm.at[pl.ds(base_row + g * ROWS_PER_CHUNK, ROWS_PER_CHUNK)],
            osems[b])
    o_descs[CHUNKS - 2].wait()
    o_descs[CHUNKS - 1].wait()


@jax.jit
def kernel(hidden_states, top_k_gates, top_k_indices):
    experts_flat = top_k_indices.reshape(-1).astype(jnp.int32)
    gates_flat = top_k_gates.reshape(-1)

    mesh = _make_mesh()
    params = pltpu.CompilerParams(needs_layout_passes=False)

    sort_fn = pl.kernel(
        _sort_kernel,
        out_type=(
            jax.ShapeDtypeStruct((NSLOT,), jnp.int32),
            jax.ShapeDtypeStruct((NSLOT,), jnp.float32),
            jax.ShapeDtypeStruct((NUM_EXPERTS,), jnp.int32),
            jax.ShapeDtypeStruct((NC * NW, NUM_EXPERTS), jnp.int32),
        ),
        mesh=mesh,
        compiler_params=params,
        scratch_types=[
            pltpu.VMEM((2 * BLK,), jnp.int32),
            pltpu.VMEM((BLK,), jnp.int32),
            pltpu.VMEM((2, NUM_EXPERTS), jnp.int32),
            pltpu.VMEM((NW, NUM_EXPERTS), jnp.int32),
            pltpu.VMEM((NUM_EXPERTS,), jnp.int32),
            pltpu.VMEM((16, 128), jnp.int32),
            pltpu.VMEM((BLK,), jnp.int32),
            pltpu.VMEM((BLK,), jnp.float32),
            (pltpu.SemaphoreType.DMA, pltpu.SemaphoreType.DMA),
        ],
    )
    token_indices, sorted_gates, tokens_per_expert, _ = sort_fn(
        experts_flat, gates_flat)

    gather_fn = pl.kernel(
        _gather_kernel,
        out_type=jax.ShapeDtypeStruct((NSLOT, D_MODEL), jnp.float32),
        mesh=mesh,
        compiler_params=params,
        scratch_types=[
            pltpu.VMEM((BLK,), jnp.int32),
            pltpu.VMEM((2, ROWS_PER_CHUNK, D_MODEL), jnp.float32),
            (pltpu.SemaphoreType.DMA, pltpu.SemaphoreType.DMA),
            (pltpu.SemaphoreType.DMA, pltpu.SemaphoreType.DMA),
        ],
    )
    sorted_hidden = gather_fn(hidden_states, token_indices)

    return sorted_hidden, tokens_per_expert, sorted_gates, token_indices
